# baseline (speedup 1.0000x reference)
import functools

import jax
import jax.numpy as jnp
from jax import lax
from jax.experimental import pallas as pl
from jax.experimental.pallas import tpu as pltpu
from jax.experimental.pallas import tpu_sc as plsc

N = 10000
E = 320000
D = 128

_info = plsc.get_sparse_core_info()
NC = _info.num_cores
NS = _info.num_subcores
NW = NC * NS
EPW = E // NW
HR = 80
NN = HR * D
RPT = NN // NS
CH = 80
NCHUNK = EPW // CH
GSZ = 25
NGRP = NCHUNK // GSZ

_mesh = plsc.VectorSubcoreMesh(core_axis_name="c", subcore_axis_name="s")


@functools.partial(
    pl.kernel,
    mesh=_mesh,
    out_type=jax.ShapeDtypeStruct((NC * NN, D), jnp.float32),
    scratch_types=[
        pltpu.VMEM((NCHUNK, CH), jnp.int32),
        pltpu.VMEM((CH, D), jnp.float32),
        pltpu.VMEM_SHARED((NN, D), jnp.float32),
        pltpu.SemaphoreType.DMA,
    ],
)
def _sc_degree(dst_hbm, ones_hbm, zeros_hbm, out_hbm,
               dst_all, ones_v, acc_sh, sem):
    c = lax.axis_index("c")
    s = lax.axis_index("s")
    wid = s * NC + c
    pltpu.sync_copy(zeros_hbm, acc_sh.at[pl.ds(s * RPT, RPT)])
    pltpu.sync_copy(dst_hbm.at[wid], dst_all)
    pltpu.sync_copy(ones_hbm, ones_v)
    plsc.subcore_barrier()

    def fire(j, carry):
        pltpu.async_copy(ones_v, acc_sh.at[dst_all.at[j]], sem, add=True)
        return carry

    lax.fori_loop(0, NCHUNK, fire, 0)

    def drain(j, carry):
        pltpu.make_async_copy(ones_v, acc_sh.at[dst_all.at[0]], sem).wait()
        return carry

    lax.fori_loop(0, NCHUNK, drain, 0)
    plsc.subcore_barrier()
    pltpu.sync_copy(acc_sh.at[pl.ds(s * RPT, RPT)],
                    out_hbm.at[pl.ds(c * NN + s * RPT, RPT)])


@functools.partial(
    pl.kernel,
    mesh=_mesh,
    out_type=jax.ShapeDtypeStruct((NC * NN, D), jnp.float32),
    scratch_types=[
        pltpu.VMEM((CH,), jnp.int32),
        pltpu.VMEM((CH,), jnp.int32),
        pltpu.VMEM((CH,), jnp.int32),
        pltpu.VMEM((CH,), jnp.int32),
        pltpu.VMEM((CH,), jnp.int32),
        pltpu.VMEM((CH,), jnp.int32),
        pltpu.VMEM((CH,), jnp.int32),
        pltpu.VMEM((CH,), jnp.int32),
        pltpu.VMEM((CH, D), jnp.float32),
        pltpu.VMEM((CH, D), jnp.float32),
        pltpu.VMEM((CH, D), jnp.float32),
        pltpu.VMEM_SHARED((NN, D), jnp.float32),
        pltpu.SemaphoreType.DMA,
        pltpu.SemaphoreType.DMA,
        pltpu.SemaphoreType.DMA,
        pltpu.SemaphoreType.DMA,
        pltpu.SemaphoreType.DMA,
        pltpu.SemaphoreType.DMA,
        pltpu.SemaphoreType.DMA,
        pltpu.SemaphoreType.DMA,
        pltpu.SemaphoreType.DMA,
        pltpu.SemaphoreType.DMA,
    ],
)
def _sc_aggregate(g_hbm, src_hbm, dst_hbm, zeros_hbm, out_hbm,
                  s0, s1, s2, s3, d0, d1, d2, d3, rows0, rows1, rows2,
                  acc_sh, ig0, ig1, ig2, ig3, gg0, gg1, gg2,
                  sg0, sg1, sg2):
    c = lax.axis_index("c")
    s = lax.axis_index("s")
    wid = s * NC + c
    ebase = wid * EPW
    pltpu.sync_copy(zeros_hbm, acc_sh.at[pl.ds(s * RPT, RPT)])
    plsc.subcore_barrier()

    sidx = (s0, s1, s2, s3)
    didx = (d0, d1, d2, d3)
    isem = (ig0, ig1, ig2, ig3)
    rows = (rows0, rows1, rows2)
    gsem = (gg0, gg1, gg2)
    ssem = (sg0, sg1, sg2)

    def load_idx(j, q):
        pltpu.async_copy(src_hbm.at[pl.ds(ebase + j * CH, CH)], sidx[q],
                         isem[q])
        pltpu.async_copy(dst_hbm.at[pl.ds(ebase + j * CH, CH)], didx[q],
                         isem[q])

    def wait_idx(q):
        pltpu.make_async_copy(src_hbm.at[pl.ds(0, CH)], sidx[q],
                              isem[q]).wait()
        pltpu.make_async_copy(src_hbm.at[pl.ds(0, CH)], didx[q],
                              isem[q]).wait()

    def wait_gather(r):
        pltpu.make_async_copy(g_hbm.at[pl.ds(0, CH)], rows[r],
                              gsem[r]).wait()

    def wait_scatter(r):
        pltpu.make_async_copy(g_hbm.at[pl.ds(0, CH)], rows[r],
                              ssem[r]).wait()

    pltpu.sync_copy(src_hbm.at[pl.ds(ebase, CH)], s0)
    pltpu.sync_copy(dst_hbm.at[pl.ds(ebase, CH)], d0)
    load_idx(1, 1)
    pltpu.async_copy(g_hbm.at[s0], rows0, gg0)
    pltpu.sync_copy(zeros_hbm.at[pl.ds(0, CH)], rows2)
    pltpu.async_copy(rows2, acc_sh.at[d0], sg1, add=True)
    pltpu.async_copy(rows2, acc_sh.at[d0], sg2, add=True)

    def step(j, r, q, tail):
        wait_gather(r)
        if tail < 2:
            wait_idx((q + 1) % 4)
        wait_scatter((r + 1) % 3)
        if tail < 2:
            pltpu.async_copy(g_hbm.at[sidx[(q + 1) % 4]], rows[(r + 1) % 3],
                             gsem[(r + 1) % 3])
        if tail < 1:
            load_idx(j + 2, (q + 2) % 4)
        pltpu.async_copy(rows[r], acc_sh.at[didx[q]], ssem[r], add=True)

    def body(i, carry):
        for b in range(12):
            step(12 * i + b, b % 3, b % 4, 0)
        return carry

    lax.fori_loop(0, (NCHUNK - 5) // 12, body, 0)
    t0 = NCHUNK - 5
    step(t0 + 0, 0, 0, 0)
    step(t0 + 1, 1, 1, 0)
    step(t0 + 2, 2, 2, 0)
    step(t0 + 3, 0, 3, 1)
    step(t0 + 4, 1, 0, 2)
    wait_scatter(0)
    wait_scatter(1)

    plsc.subcore_barrier()
    pltpu.sync_copy(acc_sh.at[pl.ds(s * RPT, RPT)],
                    out_hbm.at[pl.ds(c * NN + s * RPT, RPT)])


_BR = 1024
_GRID = NN // _BR


def _isq_idg(cnt_ref):
    deg = cnt_ref[0][:, 0:1] + cnt_ref[1][:, 0:1] + 1.0
    isq = lax.rsqrt(deg)
    return isq, 1.0 / deg


def _t1_body(x_ref, w_ref, cnt_ref, hw_ref, g_ref):
    hw = jnp.dot(x_ref[...], w_ref[...], preferred_element_type=jnp.float32)
    isq, _ = _isq_idg(cnt_ref)
    hw_ref[...] = hw
    g_ref[...] = hw * isq


def _t2_body(acc_ref, hw1_ref, cnt_ref, b1_ref, w2_ref, hw2_ref, g2_ref):
    isq, idg = _isq_idg(cnt_ref)
    hw1 = hw1_ref[...]
    agg = (acc_ref[0] + acc_ref[1]) * isq + hw1 * idg + b1_ref[...]
    h1 = jnp.maximum(agg, 0.0)
    hw2 = jnp.dot(h1, w2_ref[...], preferred_element_type=jnp.float32)
    hw2_ref[...] = hw2
    g2_ref[...] = hw2 * isq


def _t3_body(acc_ref, hw2_ref, cnt_ref, b2_ref, wo_ref, bo_ref, out_ref):
    isq, idg = _isq_idg(cnt_ref)
    h2 = (acc_ref[0] + acc_ref[1]) * isq + hw2_ref[...] * idg + b2_ref[...]
    out_ref[...] = jnp.dot(h2, wo_ref[...],
                           preferred_element_type=jnp.float32) + bo_ref[...]


def _row_spec(width):
    return pl.BlockSpec((_BR, width), lambda i: (i, 0))


_full_spec = pl.BlockSpec((D, D), lambda i: (0, 0))
_bias_spec = pl.BlockSpec((1, D), lambda i: (0, 0))
_cnt_spec = pl.BlockSpec((NC, _BR, 8), lambda i: (0, i, 0))
_acc_spec = pl.BlockSpec((2, _BR, D), lambda i: (0, i, 0))

_t1 = pl.pallas_call(
    _t1_body,
    grid=(_GRID,),
    in_specs=[_row_spec(D), _full_spec, _cnt_spec],
    out_specs=[_row_spec(D), _row_spec(D)],
    out_shape=[jax.ShapeDtypeStruct((NN, D), jnp.float32)] * 2,
)

_t2 = pl.pallas_call(
    _t2_body,
    grid=(_GRID,),
    in_specs=[_acc_spec, _row_spec(D), _cnt_spec, _bias_spec, _full_spec],
    out_specs=[_row_spec(D), _row_spec(D)],
    out_shape=[jax.ShapeDtypeStruct((NN, D), jnp.float32)] * 2,
)

_t3 = pl.pallas_call(
    _t3_body,
    grid=(_GRID,),
    in_specs=[_acc_spec, _row_spec(D), _cnt_spec, _bias_spec, _full_spec,
              _bias_spec],
    out_specs=_row_spec(D),
    out_shape=jax.ShapeDtypeStruct((NN, D), jnp.float32),
)


def kernel(x, edge_index, W1, b1, W2, b2, W_out, b_out):
    src1 = edge_index[0]
    dst1 = edge_index[1]
    dst3 = edge_index[1].reshape(NW, NCHUNK, CH)
    onesC = jnp.ones((CH, D), jnp.float32)
    zerosD = jnp.zeros((RPT, D), jnp.float32)
    xp = jnp.concatenate([x, jnp.zeros((NN - N, D), jnp.float32)], axis=0)

    cnt = _sc_degree(dst3, onesC, zerosD).reshape(NC, NN, D)[:, :, :8]

    b1r = b1.reshape(1, D)
    b2r = b2.reshape(1, D)
    wo = jnp.zeros((D, D), jnp.float32).at[:, :1].set(W_out)
    bo = jnp.zeros((1, D), jnp.float32).at[0, 0].set(b_out[0])

    hw1, g1 = _t1(xp, W1, cnt)
    acc1 = _sc_aggregate(g1, src1, dst1, zerosD).reshape(NC, NN, D)
    hw2, g2 = _t2(acc1, hw1, cnt, b1r, W2)
    acc2 = _sc_aggregate(g2, src1, dst1, zerosD).reshape(NC, NN, D)
    out = _t3(acc2, hw2, cnt, b2r, wo, bo)
    return out[:N, :1]


# speedup vs baseline: 1.0719x; 1.0719x over previous
import functools

import jax
import jax.numpy as jnp
from jax import lax
from jax.experimental import pallas as pl
from jax.experimental.pallas import tpu as pltpu
from jax.experimental.pallas import tpu_sc as plsc

N = 10000
E = 320000
D = 128

_info = plsc.get_sparse_core_info()
NC = _info.num_cores
NS = _info.num_subcores
NW = NC * NS
EPW = E // NW
HR = 80
NN = HR * D
RPT = NN // NS
CH = 80
NCHUNK = EPW // CH
GSZ = 25
NGRP = NCHUNK // GSZ

_mesh = plsc.VectorSubcoreMesh(core_axis_name="c", subcore_axis_name="s")


W64 = 64


@functools.partial(
    pl.kernel,
    mesh=_mesh,
    out_type=jax.ShapeDtypeStruct((NC * NN, W64), jnp.float32),
    scratch_types=[
        pltpu.VMEM((NCHUNK, CH), jnp.int32),
        pltpu.VMEM((CH, W64), jnp.float32),
        pltpu.VMEM((CH, W64), jnp.float32),
        pltpu.VMEM_SHARED((NN, W64), jnp.float32),
        pltpu.SemaphoreType.DMA,
    ],
)
def _sc_degree(dst_hbm, out_hbm, dst_all, ones_v, zeros_v, acc_sh, sem):
    c = lax.axis_index("c")
    s = lax.axis_index("s")
    wid = s * NC + c
    ov = jnp.ones((16,), jnp.float32)
    zv = jnp.zeros((16,), jnp.float32)

    def bld(r, carry):
        for k in range(W64 // 16):
            ones_v[r, pl.ds(k * 16, 16)] = ov
            zeros_v[r, pl.ds(k * 16, 16)] = zv
        return carry

    lax.fori_loop(0, CH, bld, 0)
    pltpu.sync_copy(dst_hbm.at[wid], dst_all)
    for k in range(RPT // CH):
        pltpu.sync_copy(zeros_v, acc_sh.at[pl.ds(s * RPT + k * CH, CH)])
    plsc.subcore_barrier()

    def fire(j, carry):
        pltpu.async_copy(ones_v, acc_sh.at[dst_all.at[j]], sem, add=True)
        return carry

    lax.fori_loop(0, NCHUNK, fire, 0)

    def drain(j, carry):
        pltpu.make_async_copy(ones_v, acc_sh.at[dst_all.at[0]], sem).wait()
        return carry

    lax.fori_loop(0, NCHUNK, drain, 0)
    plsc.subcore_barrier()
    pltpu.sync_copy(acc_sh.at[pl.ds(s * RPT, RPT)],
                    out_hbm.at[pl.ds(c * NN + s * RPT, RPT)])


@functools.partial(
    pl.kernel,
    mesh=_mesh,
    out_type=jax.ShapeDtypeStruct((NC * NN, D), jnp.float32),
    scratch_types=[
        pltpu.VMEM((CH,), jnp.int32),
        pltpu.VMEM((CH,), jnp.int32),
        pltpu.VMEM((CH,), jnp.int32),
        pltpu.VMEM((CH,), jnp.int32),
        pltpu.VMEM((CH,), jnp.int32),
        pltpu.VMEM((CH,), jnp.int32),
        pltpu.VMEM((CH,), jnp.int32),
        pltpu.VMEM((CH,), jnp.int32),
        pltpu.VMEM((CH, D), jnp.float32),
        pltpu.VMEM((CH, D), jnp.float32),
        pltpu.VMEM((CH, D), jnp.float32),
        pltpu.VMEM_SHARED((NN, D), jnp.float32),
        pltpu.SemaphoreType.DMA,
        pltpu.SemaphoreType.DMA,
        pltpu.SemaphoreType.DMA,
        pltpu.SemaphoreType.DMA,
        pltpu.SemaphoreType.DMA,
        pltpu.SemaphoreType.DMA,
        pltpu.SemaphoreType.DMA,
        pltpu.SemaphoreType.DMA,
        pltpu.SemaphoreType.DMA,
        pltpu.SemaphoreType.DMA,
    ],
)
def _sc_aggregate(g_hbm, src_hbm, dst_hbm, zeros_hbm, out_hbm,
                  s0, s1, s2, s3, d0, d1, d2, d3, rows0, rows1, rows2,
                  acc_sh, ig0, ig1, ig2, ig3, gg0, gg1, gg2,
                  sg0, sg1, sg2):
    c = lax.axis_index("c")
    s = lax.axis_index("s")
    wid = s * NC + c
    ebase = wid * EPW
    pltpu.sync_copy(zeros_hbm, acc_sh.at[pl.ds(s * RPT, RPT)])
    plsc.subcore_barrier()

    sidx = (s0, s1, s2, s3)
    didx = (d0, d1, d2, d3)
    isem = (ig0, ig1, ig2, ig3)
    rows = (rows0, rows1, rows2)
    gsem = (gg0, gg1, gg2)
    ssem = (sg0, sg1, sg2)

    def load_idx(j, q):
        pltpu.async_copy(src_hbm.at[pl.ds(ebase + j * CH, CH)], sidx[q],
                         isem[q])
        pltpu.async_copy(dst_hbm.at[pl.ds(ebase + j * CH, CH)], didx[q],
                         isem[q])

    def wait_idx(q):
        pltpu.make_async_copy(src_hbm.at[pl.ds(0, CH)], sidx[q],
                              isem[q]).wait()
        pltpu.make_async_copy(src_hbm.at[pl.ds(0, CH)], didx[q],
                              isem[q]).wait()

    def wait_gather(r):
        pltpu.make_async_copy(g_hbm.at[pl.ds(0, CH)], rows[r],
                              gsem[r]).wait()

    def wait_scatter(r):
        pltpu.make_async_copy(g_hbm.at[pl.ds(0, CH)], rows[r],
                              ssem[r]).wait()

    pltpu.sync_copy(src_hbm.at[pl.ds(ebase, CH)], s0)
    pltpu.sync_copy(dst_hbm.at[pl.ds(ebase, CH)], d0)
    load_idx(1, 1)
    pltpu.async_copy(g_hbm.at[s0], rows0, gg0)
    pltpu.sync_copy(zeros_hbm.at[pl.ds(0, CH)], rows2)
    pltpu.async_copy(rows2, acc_sh.at[d0], sg1, add=True)
    pltpu.async_copy(rows2, acc_sh.at[d0], sg2, add=True)

    def step(j, r, q, tail):
        wait_gather(r)
        if tail < 2:
            wait_idx((q + 1) % 4)
        wait_scatter((r + 1) % 3)
        if tail < 2:
            pltpu.async_copy(g_hbm.at[sidx[(q + 1) % 4]], rows[(r + 1) % 3],
                             gsem[(r + 1) % 3])
        if tail < 1:
            load_idx(j + 2, (q + 2) % 4)
        pltpu.async_copy(rows[r], acc_sh.at[didx[q]], ssem[r], add=True)

    def body(i, carry):
        for b in range(12):
            step(12 * i + b, b % 3, b % 4, 0)
        return carry

    lax.fori_loop(0, (NCHUNK - 5) // 12, body, 0)
    t0 = NCHUNK - 5
    step(t0 + 0, 0, 0, 0)
    step(t0 + 1, 1, 1, 0)
    step(t0 + 2, 2, 2, 0)
    step(t0 + 3, 0, 3, 1)
    step(t0 + 4, 1, 0, 2)
    wait_scatter(0)
    wait_scatter(1)

    plsc.subcore_barrier()
    pltpu.sync_copy(acc_sh.at[pl.ds(s * RPT, RPT)],
                    out_hbm.at[pl.ds(c * NN + s * RPT, RPT)])


_BR = 1024
_GRID = NN // _BR


def _isq_idg(cnt_ref):
    deg = cnt_ref[0][:, 0:1] + cnt_ref[1][:, 0:1] + 1.0
    isq = lax.rsqrt(deg)
    return isq, 1.0 / deg


def _t1_body(x_ref, w_ref, cnt_ref, hw_ref, g_ref):
    hw = jnp.dot(x_ref[...], w_ref[...], preferred_element_type=jnp.float32)
    isq, _ = _isq_idg(cnt_ref)
    hw_ref[...] = hw
    g_ref[...] = hw * isq


def _t2_body(acc_ref, hw1_ref, cnt_ref, b1_ref, w2_ref, hw2_ref, g2_ref):
    isq, idg = _isq_idg(cnt_ref)
    hw1 = hw1_ref[...]
    agg = (acc_ref[0] + acc_ref[1]) * isq + hw1 * idg + b1_ref[...]
    h1 = jnp.maximum(agg, 0.0)
    hw2 = jnp.dot(h1, w2_ref[...], preferred_element_type=jnp.float32)
    hw2_ref[...] = hw2
    g2_ref[...] = hw2 * isq


def _t3_body(acc_ref, hw2_ref, cnt_ref, b2_ref, wo_ref, bo_ref, out_ref):
    isq, idg = _isq_idg(cnt_ref)
    h2 = (acc_ref[0] + acc_ref[1]) * isq + hw2_ref[...] * idg + b2_ref[...]
    out_ref[...] = jnp.dot(h2, wo_ref[...],
                           preferred_element_type=jnp.float32) + bo_ref[...]


def _row_spec(width):
    return pl.BlockSpec((_BR, width), lambda i: (i, 0))


_full_spec = pl.BlockSpec((D, D), lambda i: (0, 0))
_bias_spec = pl.BlockSpec((1, D), lambda i: (0, 0))
_cnt_spec = pl.BlockSpec((NC, _BR, 8), lambda i: (0, i, 0))
_acc_spec = pl.BlockSpec((2, _BR, D), lambda i: (0, i, 0))

_t1 = pl.pallas_call(
    _t1_body,
    grid=(_GRID,),
    in_specs=[_row_spec(D), _full_spec, _cnt_spec],
    out_specs=[_row_spec(D), _row_spec(D)],
    out_shape=[jax.ShapeDtypeStruct((NN, D), jnp.float32)] * 2,
)

_t2 = pl.pallas_call(
    _t2_body,
    grid=(_GRID,),
    in_specs=[_acc_spec, _row_spec(D), _cnt_spec, _bias_spec, _full_spec],
    out_specs=[_row_spec(D), _row_spec(D)],
    out_shape=[jax.ShapeDtypeStruct((NN, D), jnp.float32)] * 2,
)

_t3 = pl.pallas_call(
    _t3_body,
    grid=(_GRID,),
    in_specs=[_acc_spec, _row_spec(D), _cnt_spec, _bias_spec, _full_spec,
              _bias_spec],
    out_specs=_row_spec(D),
    out_shape=jax.ShapeDtypeStruct((NN, D), jnp.float32),
)


def kernel(x, edge_index, W1, b1, W2, b2, W_out, b_out):
    src1 = edge_index[0]
    dst1 = edge_index[1]
    dst3 = edge_index[1].reshape(NW, NCHUNK, CH)
    zerosD = jnp.zeros((RPT, D), jnp.float32)
    xp = jnp.concatenate([x, jnp.zeros((NN - N, D), jnp.float32)], axis=0)

    cnt = _sc_degree(dst3).reshape(NC, NN, W64)[:, :, :8]

    b1r = b1.reshape(1, D)
    b2r = b2.reshape(1, D)
    wo = jnp.zeros((D, D), jnp.float32).at[:, :1].set(W_out)
    bo = jnp.zeros((1, D), jnp.float32).at[0, 0].set(b_out[0])

    hw1, g1 = _t1(xp, W1, cnt)
    acc1 = _sc_aggregate(g1, src1, dst1, zerosD).reshape(NC, NN, D)
    hw2, g2 = _t2(acc1, hw1, cnt, b1r, W2)
    acc2 = _sc_aggregate(g2, src1, dst1, zerosD).reshape(NC, NN, D)
    out = _t3(acc2, hw2, cnt, b2r, wo, bo)
    return out[:N, :1]


# speedup vs baseline: 1.1022x; 1.0283x over previous
import functools

import jax
import jax.numpy as jnp
from jax import lax
from jax.experimental import pallas as pl
from jax.experimental.pallas import tpu as pltpu
from jax.experimental.pallas import tpu_sc as plsc

N = 10000
E = 320000
D = 128

_info = plsc.get_sparse_core_info()
NC = _info.num_cores
NS = _info.num_subcores
NW = NC * NS
EPW = E // NW
HR = 80
NN = HR * D
RPT = NN // NS
CH = 80
NCHUNK = EPW // CH
GSZ = 25
NGRP = NCHUNK // GSZ

_mesh = plsc.VectorSubcoreMesh(core_axis_name="c", subcore_axis_name="s")


W64 = 32


@functools.partial(
    pl.kernel,
    mesh=_mesh,
    out_type=jax.ShapeDtypeStruct((NC * NN, W64), jnp.float32),
    scratch_types=[
        pltpu.VMEM((NCHUNK, CH), jnp.int32),
        pltpu.VMEM((CH, W64), jnp.float32),
        pltpu.VMEM((CH, W64), jnp.float32),
        pltpu.VMEM_SHARED((NN, W64), jnp.float32),
        pltpu.SemaphoreType.DMA,
    ],
)
def _sc_degree(dst_hbm, out_hbm, dst_all, ones_v, zeros_v, acc_sh, sem):
    c = lax.axis_index("c")
    s = lax.axis_index("s")
    wid = s * NC + c
    ov = jnp.ones((16,), jnp.float32)
    zv = jnp.zeros((16,), jnp.float32)

    def bld(r, carry):
        for k in range(W64 // 16):
            ones_v[r, pl.ds(k * 16, 16)] = ov
            zeros_v[r, pl.ds(k * 16, 16)] = zv
        return carry

    lax.fori_loop(0, CH, bld, 0)
    pltpu.sync_copy(dst_hbm.at[wid], dst_all)
    for k in range(RPT // CH):
        pltpu.sync_copy(zeros_v, acc_sh.at[pl.ds(s * RPT + k * CH, CH)])
    plsc.subcore_barrier()

    def fire(j, carry):
        pltpu.async_copy(ones_v, acc_sh.at[dst_all.at[j]], sem, add=True)
        return carry

    lax.fori_loop(0, NCHUNK, fire, 0)

    def drain(j, carry):
        pltpu.make_async_copy(ones_v, acc_sh.at[dst_all.at[0]], sem).wait()
        return carry

    lax.fori_loop(0, NCHUNK, drain, 0)
    plsc.subcore_barrier()
    pltpu.sync_copy(acc_sh.at[pl.ds(s * RPT, RPT)],
                    out_hbm.at[pl.ds(c * NN + s * RPT, RPT)])


@functools.partial(
    pl.kernel,
    mesh=_mesh,
    out_type=jax.ShapeDtypeStruct((NC * NN, D), jnp.float32),
    scratch_types=[
        pltpu.VMEM((CH,), jnp.int32),
        pltpu.VMEM((CH,), jnp.int32),
        pltpu.VMEM((CH,), jnp.int32),
        pltpu.VMEM((CH,), jnp.int32),
        pltpu.VMEM((CH,), jnp.int32),
        pltpu.VMEM((CH,), jnp.int32),
        pltpu.VMEM((CH,), jnp.int32),
        pltpu.VMEM((CH,), jnp.int32),
        pltpu.VMEM((CH, D), jnp.float32),
        pltpu.VMEM((CH, D), jnp.float32),
        pltpu.VMEM((CH, D), jnp.float32),
        pltpu.VMEM_SHARED((NN, D), jnp.float32),
        pltpu.SemaphoreType.DMA,
        pltpu.SemaphoreType.DMA,
        pltpu.SemaphoreType.DMA,
        pltpu.SemaphoreType.DMA,
        pltpu.SemaphoreType.DMA,
        pltpu.SemaphoreType.DMA,
        pltpu.SemaphoreType.DMA,
        pltpu.SemaphoreType.DMA,
        pltpu.SemaphoreType.DMA,
        pltpu.SemaphoreType.DMA,
    ],
)
def _sc_aggregate(g_hbm, src_hbm, dst_hbm, zeros_hbm, out_hbm,
                  s0, s1, s2, s3, d0, d1, d2, d3, rows0, rows1, rows2,
                  acc_sh, ig0, ig1, ig2, ig3, gg0, gg1, gg2,
                  sg0, sg1, sg2):
    c = lax.axis_index("c")
    s = lax.axis_index("s")
    wid = s * NC + c
    ebase = wid * EPW
    pltpu.sync_copy(zeros_hbm, acc_sh.at[pl.ds(s * RPT, RPT)])
    plsc.subcore_barrier()

    sidx = (s0, s1, s2, s3)
    didx = (d0, d1, d2, d3)
    isem = (ig0, ig1, ig2, ig3)
    rows = (rows0, rows1, rows2)
    gsem = (gg0, gg1, gg2)
    ssem = (sg0, sg1, sg2)

    def load_idx(j, q):
        pltpu.async_copy(src_hbm.at[pl.ds(ebase + j * CH, CH)], sidx[q],
                         isem[q])
        pltpu.async_copy(dst_hbm.at[pl.ds(ebase + j * CH, CH)], didx[q],
                         isem[q])

    def wait_idx(q):
        pltpu.make_async_copy(src_hbm.at[pl.ds(0, CH)], sidx[q],
                              isem[q]).wait()
        pltpu.make_async_copy(src_hbm.at[pl.ds(0, CH)], didx[q],
                              isem[q]).wait()

    def wait_gather(r):
        pltpu.make_async_copy(g_hbm.at[pl.ds(0, CH)], rows[r],
                              gsem[r]).wait()

    def wait_scatter(r):
        pltpu.make_async_copy(g_hbm.at[pl.ds(0, CH)], rows[r],
                              ssem[r]).wait()

    pltpu.sync_copy(src_hbm.at[pl.ds(ebase, CH)], s0)
    pltpu.sync_copy(dst_hbm.at[pl.ds(ebase, CH)], d0)
    load_idx(1, 1)
    pltpu.async_copy(g_hbm.at[s0], rows0, gg0)
    pltpu.sync_copy(zeros_hbm.at[pl.ds(0, CH)], rows2)
    pltpu.async_copy(rows2, acc_sh.at[d0], sg1, add=True)
    pltpu.async_copy(rows2, acc_sh.at[d0], sg2, add=True)

    def step(j, r, q, tail):
        wait_gather(r)
        if tail < 2:
            wait_idx((q + 1) % 4)
        wait_scatter((r + 1) % 3)
        if tail < 2:
            pltpu.async_copy(g_hbm.at[sidx[(q + 1) % 4]], rows[(r + 1) % 3],
                             gsem[(r + 1) % 3])
        if tail < 1:
            load_idx(j + 2, (q + 2) % 4)
        pltpu.async_copy(rows[r], acc_sh.at[didx[q]], ssem[r], add=True)

    def body(i, carry):
        for b in range(12):
            step(12 * i + b, b % 3, b % 4, 0)
        return carry

    lax.fori_loop(0, (NCHUNK - 5) // 12, body, 0)
    t0 = NCHUNK - 5
    step(t0 + 0, 0, 0, 0)
    step(t0 + 1, 1, 1, 0)
    step(t0 + 2, 2, 2, 0)
    step(t0 + 3, 0, 3, 1)
    step(t0 + 4, 1, 0, 2)
    wait_scatter(0)
    wait_scatter(1)

    plsc.subcore_barrier()
    pltpu.sync_copy(acc_sh.at[pl.ds(s * RPT, RPT)],
                    out_hbm.at[pl.ds(c * NN + s * RPT, RPT)])


_BR = 1024
_GRID = NN // _BR


def _isq_idg(cnt_ref):
    deg = cnt_ref[0][:, 0:1] + cnt_ref[1][:, 0:1] + 1.0
    isq = lax.rsqrt(deg)
    return isq, 1.0 / deg


def _t1_body(x_ref, w_ref, cnt_ref, hw_ref, g_ref):
    hw = jnp.dot(x_ref[...], w_ref[...], preferred_element_type=jnp.float32)
    isq, _ = _isq_idg(cnt_ref)
    hw_ref[...] = hw
    g_ref[...] = hw * isq


def _t2_body(acc_ref, hw1_ref, cnt_ref, b1_ref, w2_ref, hw2_ref, g2_ref):
    isq, idg = _isq_idg(cnt_ref)
    hw1 = hw1_ref[...]
    agg = (acc_ref[0] + acc_ref[1]) * isq + hw1 * idg + b1_ref[...]
    h1 = jnp.maximum(agg, 0.0)
    hw2 = jnp.dot(h1, w2_ref[...], preferred_element_type=jnp.float32)
    hw2_ref[...] = hw2
    g2_ref[...] = hw2 * isq


def _t3_body(acc_ref, hw2_ref, cnt_ref, b2_ref, wo_ref, bo_ref, out_ref):
    isq, idg = _isq_idg(cnt_ref)
    h2 = (acc_ref[0] + acc_ref[1]) * isq + hw2_ref[...] * idg + b2_ref[...]
    out_ref[...] = jnp.dot(h2, wo_ref[...],
                           preferred_element_type=jnp.float32) + bo_ref[...]


def _row_spec(width):
    return pl.BlockSpec((_BR, width), lambda i: (i, 0))


_full_spec = pl.BlockSpec((D, D), lambda i: (0, 0))
_bias_spec = pl.BlockSpec((1, D), lambda i: (0, 0))
_cnt_spec = pl.BlockSpec((NC, _BR, 8), lambda i: (0, i, 0))
_acc_spec = pl.BlockSpec((2, _BR, D), lambda i: (0, i, 0))

_t1 = pl.pallas_call(
    _t1_body,
    grid=(_GRID,),
    in_specs=[_row_spec(D), _full_spec, _cnt_spec],
    out_specs=[_row_spec(D), _row_spec(D)],
    out_shape=[jax.ShapeDtypeStruct((NN, D), jnp.float32)] * 2,
)

_t2 = pl.pallas_call(
    _t2_body,
    grid=(_GRID,),
    in_specs=[_acc_spec, _row_spec(D), _cnt_spec, _bias_spec, _full_spec],
    out_specs=[_row_spec(D), _row_spec(D)],
    out_shape=[jax.ShapeDtypeStruct((NN, D), jnp.float32)] * 2,
)

_t3 = pl.pallas_call(
    _t3_body,
    grid=(_GRID,),
    in_specs=[_acc_spec, _row_spec(D), _cnt_spec, _bias_spec, _full_spec,
              _bias_spec],
    out_specs=_row_spec(D),
    out_shape=jax.ShapeDtypeStruct((NN, D), jnp.float32),
)


def kernel(x, edge_index, W1, b1, W2, b2, W_out, b_out):
    src1 = edge_index[0]
    dst1 = edge_index[1]
    dst3 = edge_index[1].reshape(NW, NCHUNK, CH)
    zerosD = jnp.zeros((RPT, D), jnp.float32)
    xp = jnp.concatenate([x, jnp.zeros((NN - N, D), jnp.float32)], axis=0)

    cnt = _sc_degree(dst3).reshape(NC, NN, W64)[:, :, :8]

    b1r = b1.reshape(1, D)
    b2r = b2.reshape(1, D)
    wo = jnp.zeros((D, D), jnp.float32).at[:, :1].set(W_out)
    bo = jnp.zeros((1, D), jnp.float32).at[0, 0].set(b_out[0])

    hw1, g1 = _t1(xp, W1, cnt)
    acc1 = _sc_aggregate(g1, src1, dst1, zerosD).reshape(NC, NN, D)
    hw2, g2 = _t2(acc1, hw1, cnt, b1r, W2)
    acc2 = _sc_aggregate(g2, src1, dst1, zerosD).reshape(NC, NN, D)
    out = _t3(acc2, hw2, cnt, b2r, wo, bo)
    return out[:N, :1]
